# baseline (speedup 1.0000x reference)
import functools

import jax
import jax.numpy as jnp
from jax import lax
from jax.experimental import pallas as pl
from jax.experimental.pallas import tpu as pltpu
from jax.experimental.pallas import tpu_sc as plsc

N = 10000
E = 160000
H = 128
NC, NS = 2, 16
NW = NC * NS
CH = 128
NCHUNK = E // CH
ROWS_PER_TILE = 624
TAIL_ROWS = N - NS * ROWS_PER_TILE

BN = 2000
BE = 2000

_f32 = jnp.float32


def _full(shape):
    return pl.BlockSpec(shape, lambda i: (0,) * len(shape))


def _rows(block):
    return pl.BlockSpec(block, lambda i: (i,) + (0,) * (len(block) - 1))


def _ln(x, g, be):
    mu = jnp.mean(x, axis=-1, keepdims=True)
    var = jnp.var(x, axis=-1, keepdims=True)
    return (x - mu) / jnp.sqrt(var + 1e-5) * g + be


def _node_enc_body(x_ref, pos_ref, embp_ref, w1_ref, b1_ref,
                   w2_ref, b2_ref, w3_ref, b3_ref, h_ref):
    oh = (x_ref[...] == lax.broadcasted_iota(jnp.int32, (BN, 16), 1)).astype(_f32)
    e = jnp.dot(oh, embp_ref[...], preferred_element_type=_f32,
                precision=lax.Precision.HIGHEST)
    cat = jnp.concatenate([e, pos_ref[...]], axis=1)
    h = jnp.maximum(jnp.dot(cat, w1_ref[...], preferred_element_type=_f32)
                    + b1_ref[...], 0.0)
    h = jnp.maximum(jnp.dot(h, w2_ref[...], preferred_element_type=_f32)
                    + b2_ref[...], 0.0)
    h_ref[...] = jnp.dot(h, w3_ref[...], preferred_element_type=_f32) + b3_ref[...]


def _node_enc(x2d, pos, embp, w1, b1, w2, b2, w3, b3):
    return pl.pallas_call(
        _node_enc_body,
        grid=(N // BN,),
        in_specs=[
            _rows((BN, 1)), _rows((BN, 14)),
            _full((16, 16)), _full((30, H)), _full((1, H)),
            _full((H, H)), _full((1, H)), _full((H, H)), _full((1, H)),
        ],
        out_specs=[_rows((BN, H))],
        out_shape=[jax.ShapeDtypeStruct((N, H), _f32)],
    )(x2d, pos, embp, w1, b1, w2, b2, w3, b3)[0]


def _edge_enc_body(ea_ref, w1_ref, b1_ref, w2_ref, b2_ref, w3_ref, b3_ref, h_ref):
    h = jnp.maximum(jnp.dot(ea_ref[...], w1_ref[...], preferred_element_type=_f32)
                    + b1_ref[...], 0.0)
    h = jnp.maximum(jnp.dot(h, w2_ref[...], preferred_element_type=_f32)
                    + b2_ref[...], 0.0)
    h_ref[...] = jnp.dot(h, w3_ref[...], preferred_element_type=_f32) + b3_ref[...]


def _edge_enc(ea, w1, b1, w2, b2, w3, b3):
    return pl.pallas_call(
        _edge_enc_body,
        grid=(E // BE,),
        in_specs=[
            _rows((BE, 3)),
            _full((3, H)), _full((1, H)), _full((H, H)), _full((1, H)),
            _full((H, H)), _full((1, H)),
        ],
        out_specs=[_rows((BE, H))],
        out_shape=[jax.ShapeDtypeStruct((E, H), _f32)],
    )(ea, w1, b1, w2, b2, w3, b3)[0]


def _edge_dots_body(ga_ref, gb_ref, ef_ref, w1_ref, b1_ref,
                    w2_ref, b2_ref, w3_ref, b3_ref, h_ref):
    cat = jnp.concatenate([ga_ref[...], gb_ref[...], ef_ref[...]], axis=1)
    u = jnp.maximum(jnp.dot(cat, w1_ref[...], preferred_element_type=_f32)
                    + b1_ref[...], 0.0)
    h = jnp.maximum(jnp.dot(u, w2_ref[...], preferred_element_type=_f32)
                    + b2_ref[...], 0.0)
    h_ref[...] = jnp.dot(h, w3_ref[...], preferred_element_type=_f32) + b3_ref[...]


def _edge_dots(ga, gb, ef, w1, b1, w2, b2, w3, b3):
    return pl.pallas_call(
        _edge_dots_body,
        grid=(E // BE,),
        in_specs=[
            _rows((BE, H)), _rows((BE, H)), _rows((BE, H)),
            _full((3 * H, H)), _full((1, H)), _full((H, H)), _full((1, H)),
            _full((H, H)), _full((1, H)),
        ],
        out_specs=[_rows((BE, H))],
        out_shape=[jax.ShapeDtypeStruct((E, H), _f32)],
    )(ga, gb, ef, w1, b1, w2, b2, w3, b3)[0]


def _node_dots_body(nf_ref, ag_ref, w1_ref, b1_ref,
                    w2_ref, b2_ref, w3_ref, b3_ref, h_ref):
    cat = jnp.concatenate([nf_ref[...], ag_ref[...]], axis=1)
    u = jnp.maximum(jnp.dot(cat, w1_ref[...], preferred_element_type=_f32)
                    + b1_ref[...], 0.0)
    h = jnp.maximum(jnp.dot(u, w2_ref[...], preferred_element_type=_f32)
                    + b2_ref[...], 0.0)
    h_ref[...] = jnp.dot(h, w3_ref[...], preferred_element_type=_f32) + b3_ref[...]


def _node_dots(nf, aggr, w1, b1, w2, b2, w3, b3):
    return pl.pallas_call(
        _node_dots_body,
        grid=(N // BN,),
        in_specs=[
            _rows((BN, H)), _rows((BN, H)),
            _full((2 * H, H)), _full((1, H)), _full((H, H)), _full((1, H)),
            _full((H, H)), _full((1, H)),
        ],
        out_specs=[_rows((BN, H))],
        out_shape=[jax.ShapeDtypeStruct((N, H), _f32)],
    )(nf, aggr, w1, b1, w2, b2, w3, b3)[0]


def _decoder_body(nf_ref, w1_ref, b1_ref, w2_ref, b2_ref, w3_ref, b3_ref, out_ref):
    h = jnp.maximum(jnp.dot(nf_ref[...], w1_ref[...], preferred_element_type=_f32)
                    + b1_ref[...], 0.0)
    h = jnp.maximum(jnp.dot(h, w2_ref[...], preferred_element_type=_f32)
                    + b2_ref[...], 0.0)
    out_ref[...] = jnp.dot(h, w3_ref[...], preferred_element_type=_f32) + b3_ref[...]


def _decoder(nf, w1, b1, w2, b2, w3, b3):
    return pl.pallas_call(
        _decoder_body,
        grid=(N // BN,),
        in_specs=[
            _rows((BN, H)),
            _full((H, H)), _full((1, H)), _full((H, H)), _full((1, H)),
            _full((H, 2)), _full((1, 2)),
        ],
        out_specs=[_rows((BN, 2))],
        out_shape=[jax.ShapeDtypeStruct((N, 2), _f32)],
    )(nf, w1, b1, w2, b2, w3, b3)[0]


@functools.lru_cache(maxsize=None)
def _get_sc_kernels():
    mesh = plsc.VectorSubcoreMesh(core_axis_name="c", subcore_axis_name="s",
                                  num_cores=NC, num_subcores=NS)

    @functools.partial(
        pl.kernel,
        mesh=mesh,
        out_type=[jax.ShapeDtypeStruct((E, H), _f32)] * 2,
        scratch_types=[
            pltpu.VMEM((CH,), jnp.int32),
            pltpu.VMEM((CH,), jnp.int32),
            pltpu.VMEM((CH, H), _f32),
            pltpu.VMEM((CH, H), _f32),
            pltpu.SemaphoreType.DMA,
            pltpu.SemaphoreType.DMA,
        ],
    )
    def sc_gather2(a_hbm, b_hbm, dst_hbm, src_hbm, oa_hbm, ob_hbm,
                   idx_a, idx_b, buf_a, buf_b, sem_a, sem_b):
        wid = lax.axis_index("s") * NC + lax.axis_index("c")
        nch = 39 + (wid < NCHUNK - 39 * NW).astype(jnp.int32)

        def chunk(i, _):
            off = (wid + i * NW) * CH
            pltpu.sync_copy(dst_hbm.at[pl.ds(off, CH)], idx_a)
            pltpu.sync_copy(src_hbm.at[pl.ds(off, CH)], idx_b)
            cp_a = pltpu.async_copy(a_hbm.at[idx_a], buf_a, sem_a)
            cp_b = pltpu.async_copy(b_hbm.at[idx_b], buf_b, sem_b)
            cp_a.wait()
            cp_b.wait()
            pltpu.sync_copy(buf_a, oa_hbm.at[pl.ds(off, CH)])
            pltpu.sync_copy(buf_b, ob_hbm.at[pl.ds(off, CH)])
            return _

        lax.fori_loop(0, nch, chunk, None)

    @functools.partial(
        pl.kernel,
        mesh=mesh,
        out_type=[jax.ShapeDtypeStruct((NC * N, H), _f32)],
        scratch_types=[
            pltpu.VMEM((CH,), jnp.int32),
            pltpu.VMEM((CH, H), _f32),
            pltpu.VMEM_SHARED((N, H), _f32),
        ],
    )
    def sc_scatter_add(wm_hbm, dst_hbm, zeros_hbm, out_hbm, idx_v, buf_v, accum):
        cid = lax.axis_index("c")
        sid = lax.axis_index("s")
        wid = sid * NC + cid
        row0 = sid * ROWS_PER_TILE
        pltpu.sync_copy(zeros_hbm.at[pl.ds(row0, ROWS_PER_TILE)],
                        accum.at[pl.ds(row0, ROWS_PER_TILE)])

        @pl.when(sid == NS - 1)
        def _zero_tail():
            pltpu.sync_copy(zeros_hbm.at[pl.ds(NS * ROWS_PER_TILE, TAIL_ROWS)],
                            accum.at[pl.ds(NS * ROWS_PER_TILE, TAIL_ROWS)])

        plsc.subcore_barrier()

        nch = jnp.where(wid == 0, NCHUNK, 0)

        def chunk(i, _):
            off = i * CH
            pltpu.sync_copy(dst_hbm.at[pl.ds(off, CH)], idx_v)
            pltpu.sync_copy(wm_hbm.at[pl.ds(off, CH)], buf_v)
            pltpu.sync_copy(buf_v, accum.at[idx_v], add=True)
            return _

        lax.fori_loop(0, nch, chunk, None)
        plsc.subcore_barrier()
        pltpu.sync_copy(accum.at[pl.ds(row0, ROWS_PER_TILE)],
                        out_hbm.at[pl.ds(cid * N + row0, ROWS_PER_TILE)])

        @pl.when(sid == NS - 1)
        def _flush_tail():
            pltpu.sync_copy(
                accum.at[pl.ds(NS * ROWS_PER_TILE, TAIL_ROWS)],
                out_hbm.at[pl.ds(cid * N + NS * ROWS_PER_TILE, TAIL_ROWS)])

    return sc_gather2, sc_scatter_add


def _sc_gather2(a, b, dst, src):
    return _get_sc_kernels()[0](a, b, dst, src)


def _sc_scatter_add(wm, dst, zeros_nh):
    return _get_sc_kernels()[1](wm, dst, zeros_nh)


def kernel(x, pos, edge_index, edge_attr, node_dist, params):
    x2d = x.astype(jnp.int32).reshape(N, 1)
    dst = edge_index[1].astype(jnp.int32)
    src = edge_index[0].astype(jnp.int32)

    def lins(p):
        out = []
        for w, b in p["lins"]:
            out.append(w)
            out.append(b.reshape(1, -1))
        return out

    embp = jnp.zeros((16, 16), _f32).at[:9].set(params["embed"])

    nf = _ln(_node_enc(x2d, pos, embp, *lins(params["node_in"])),
             *params["node_in"]["ln"])
    ef = _ln(_edge_enc(edge_attr, *lins(params["edge_in"])),
             *params["edge_in"]["ln"])

    zeros_nh = jnp.zeros((N, H), _f32)

    for lp in params["mp"]:
        ga, gb = _sc_gather2(nf, nf, dst, src)
        msg = _ln(_edge_dots(ga, gb, ef, *lins(lp["lin_edge"])),
                  *lp["lin_edge"]["ln"])
        parts, = _sc_scatter_add(msg * node_dist, dst, zeros_nh)
        aggr = parts[:N] + parts[N:]
        upd = _ln(_node_dots(nf, aggr, *lins(lp["lin_node"])),
                  *lp["lin_node"]["ln"])
        ef = ef + msg
        nf = nf + upd

    return _decoder(nf, *lins(params["node_out"]))


# speedup vs baseline: 3.6154x; 3.6154x over previous
import functools

import jax
import jax.numpy as jnp
from jax import lax
from jax.experimental import pallas as pl
from jax.experimental.pallas import tpu as pltpu
from jax.experimental.pallas import tpu_sc as plsc

N = 10000
E = 160000
H = 128
NC, NS = 2, 16
NW = NC * NS
CH = 128
NCHUNK = E // CH
OWN_ROWS = 312
OWN_TAIL = N - NW * OWN_ROWS
LPAD = E + NW * CH

BN = 2000
BE = 2000

_f32 = jnp.float32


def _full(shape):
    return pl.BlockSpec(shape, lambda i: (0,) * len(shape))


def _rows(block):
    return pl.BlockSpec(block, lambda i: (i,) + (0,) * (len(block) - 1))


def _ln(x, g, be):
    mu = jnp.mean(x, axis=-1, keepdims=True)
    var = jnp.var(x, axis=-1, keepdims=True)
    return (x - mu) / jnp.sqrt(var + 1e-5) * g + be


def _node_enc_body(x_ref, pos_ref, embp_ref, w1_ref, b1_ref,
                   w2_ref, b2_ref, w3_ref, b3_ref, h_ref):
    oh = (x_ref[...] == lax.broadcasted_iota(jnp.int32, (BN, 16), 1)).astype(_f32)
    e = jnp.dot(oh, embp_ref[...], preferred_element_type=_f32,
                precision=lax.Precision.HIGHEST)
    cat = jnp.concatenate([e, pos_ref[...]], axis=1)
    h = jnp.maximum(jnp.dot(cat, w1_ref[...], preferred_element_type=_f32)
                    + b1_ref[...], 0.0)
    h = jnp.maximum(jnp.dot(h, w2_ref[...], preferred_element_type=_f32)
                    + b2_ref[...], 0.0)
    h_ref[...] = jnp.dot(h, w3_ref[...], preferred_element_type=_f32) + b3_ref[...]


def _node_enc(x2d, pos, embp, w1, b1, w2, b2, w3, b3):
    return pl.pallas_call(
        _node_enc_body,
        grid=(N // BN,),
        in_specs=[
            _rows((BN, 1)), _rows((BN, 14)),
            _full((16, 16)), _full((30, H)), _full((1, H)),
            _full((H, H)), _full((1, H)), _full((H, H)), _full((1, H)),
        ],
        out_specs=[_rows((BN, H))],
        out_shape=[jax.ShapeDtypeStruct((N, H), _f32)],
    )(x2d, pos, embp, w1, b1, w2, b2, w3, b3)[0]


def _edge_enc_body(ea_ref, w1_ref, b1_ref, w2_ref, b2_ref, w3_ref, b3_ref, h_ref):
    h = jnp.maximum(jnp.dot(ea_ref[...], w1_ref[...], preferred_element_type=_f32)
                    + b1_ref[...], 0.0)
    h = jnp.maximum(jnp.dot(h, w2_ref[...], preferred_element_type=_f32)
                    + b2_ref[...], 0.0)
    h_ref[...] = jnp.dot(h, w3_ref[...], preferred_element_type=_f32) + b3_ref[...]


def _edge_enc(ea, w1, b1, w2, b2, w3, b3):
    return pl.pallas_call(
        _edge_enc_body,
        grid=(E // BE,),
        in_specs=[
            _rows((BE, 3)),
            _full((3, H)), _full((1, H)), _full((H, H)), _full((1, H)),
            _full((H, H)), _full((1, H)),
        ],
        out_specs=[_rows((BE, H))],
        out_shape=[jax.ShapeDtypeStruct((E, H), _f32)],
    )(ea, w1, b1, w2, b2, w3, b3)[0]


def _edge_dots_body(ga_ref, gb_ref, ef_ref, w1_ref, b1_ref,
                    w2_ref, b2_ref, w3_ref, b3_ref, h_ref):
    cat = jnp.concatenate([ga_ref[...], gb_ref[...], ef_ref[...]], axis=1)
    u = jnp.maximum(jnp.dot(cat, w1_ref[...], preferred_element_type=_f32)
                    + b1_ref[...], 0.0)
    h = jnp.maximum(jnp.dot(u, w2_ref[...], preferred_element_type=_f32)
                    + b2_ref[...], 0.0)
    h_ref[...] = jnp.dot(h, w3_ref[...], preferred_element_type=_f32) + b3_ref[...]


def _edge_dots(ga, gb, ef, w1, b1, w2, b2, w3, b3):
    return pl.pallas_call(
        _edge_dots_body,
        grid=(E // BE,),
        in_specs=[
            _rows((BE, H)), _rows((BE, H)), _rows((BE, H)),
            _full((3 * H, H)), _full((1, H)), _full((H, H)), _full((1, H)),
            _full((H, H)), _full((1, H)),
        ],
        out_specs=[_rows((BE, H))],
        out_shape=[jax.ShapeDtypeStruct((E, H), _f32)],
    )(ga, gb, ef, w1, b1, w2, b2, w3, b3)[0]


def _node_dots_body(nf_ref, ag_ref, w1_ref, b1_ref,
                    w2_ref, b2_ref, w3_ref, b3_ref, h_ref):
    cat = jnp.concatenate([nf_ref[...], ag_ref[...]], axis=1)
    u = jnp.maximum(jnp.dot(cat, w1_ref[...], preferred_element_type=_f32)
                    + b1_ref[...], 0.0)
    h = jnp.maximum(jnp.dot(u, w2_ref[...], preferred_element_type=_f32)
                    + b2_ref[...], 0.0)
    h_ref[...] = jnp.dot(h, w3_ref[...], preferred_element_type=_f32) + b3_ref[...]


def _node_dots(nf, aggr, w1, b1, w2, b2, w3, b3):
    return pl.pallas_call(
        _node_dots_body,
        grid=(N // BN,),
        in_specs=[
            _rows((BN, H)), _rows((BN, H)),
            _full((2 * H, H)), _full((1, H)), _full((H, H)), _full((1, H)),
            _full((H, H)), _full((1, H)),
        ],
        out_specs=[_rows((BN, H))],
        out_shape=[jax.ShapeDtypeStruct((N, H), _f32)],
    )(nf, aggr, w1, b1, w2, b2, w3, b3)[0]


def _decoder_body(nf_ref, w1_ref, b1_ref, w2_ref, b2_ref, w3_ref, b3_ref, out_ref):
    h = jnp.maximum(jnp.dot(nf_ref[...], w1_ref[...], preferred_element_type=_f32)
                    + b1_ref[...], 0.0)
    h = jnp.maximum(jnp.dot(h, w2_ref[...], preferred_element_type=_f32)
                    + b2_ref[...], 0.0)
    out_ref[...] = jnp.dot(h, w3_ref[...], preferred_element_type=_f32) + b3_ref[...]


def _decoder(nf, w1, b1, w2, b2, w3, b3):
    return pl.pallas_call(
        _decoder_body,
        grid=(N // BN,),
        in_specs=[
            _rows((BN, H)),
            _full((H, H)), _full((1, H)), _full((H, H)), _full((1, H)),
            _full((H, 2)), _full((1, 2)),
        ],
        out_specs=[_rows((BN, 2))],
        out_shape=[jax.ShapeDtypeStruct((N, 2), _f32)],
    )(nf, w1, b1, w2, b2, w3, b3)[0]


@functools.lru_cache(maxsize=None)
def _get_sc_kernels():
    mesh = plsc.VectorSubcoreMesh(core_axis_name="c", subcore_axis_name="s",
                                  num_cores=NC, num_subcores=NS)

    @functools.partial(
        pl.kernel,
        mesh=mesh,
        out_type=[jax.ShapeDtypeStruct((E, H), _f32)] * 2,
        scratch_types=[
            pltpu.VMEM((CH,), jnp.int32),
            pltpu.VMEM((CH,), jnp.int32),
            pltpu.VMEM((CH, H), _f32),
            pltpu.VMEM((CH, H), _f32),
            pltpu.SemaphoreType.DMA,
            pltpu.SemaphoreType.DMA,
        ],
    )
    def sc_gather2(a_hbm, b_hbm, dst_hbm, src_hbm, oa_hbm, ob_hbm,
                   idx_a, idx_b, buf_a, buf_b, sem_a, sem_b):
        wid = lax.axis_index("s") * NC + lax.axis_index("c")
        nch = 39 + (wid < NCHUNK - 39 * NW).astype(jnp.int32)

        def chunk(i, _):
            off = (wid + i * NW) * CH
            pltpu.sync_copy(dst_hbm.at[pl.ds(off, CH)], idx_a)
            pltpu.sync_copy(src_hbm.at[pl.ds(off, CH)], idx_b)
            cp_a = pltpu.async_copy(a_hbm.at[idx_a], buf_a, sem_a)
            cp_b = pltpu.async_copy(b_hbm.at[idx_b], buf_b, sem_b)
            cp_a.wait()
            cp_b.wait()
            pltpu.sync_copy(buf_a, oa_hbm.at[pl.ds(off, CH)])
            pltpu.sync_copy(buf_b, ob_hbm.at[pl.ds(off, CH)])
            return _

        lax.fori_loop(0, nch, chunk, None)

    @functools.partial(
        pl.kernel,
        mesh=mesh,
        out_type=[jax.ShapeDtypeStruct((N, H), _f32)],
        scratch_types=[
            pltpu.VMEM((NW,), jnp.int32),
            pltpu.VMEM((NW,), jnp.int32),
            pltpu.VMEM((CH,), jnp.int32),
            pltpu.VMEM((CH,), jnp.int32),
            pltpu.VMEM((CH, H), _f32),
            pltpu.VMEM_SHARED((N + 8, H), _f32),
            pltpu.SemaphoreType.DMA,
        ],
    )
    def sc_scatter_add(wm_hbm, eid_hbm, dstp_hbm, pch_hbm, pst_hbm, zeros_hbm,
                       out_hbm, pch_v, pst_v, eid_v, idx_v, buf_v, accum, sem):
        cid = lax.axis_index("c")
        sid = lax.axis_index("s")
        wid = sid * NC + cid
        row0 = wid * OWN_ROWS
        pltpu.sync_copy(zeros_hbm.at[pl.ds(row0, OWN_ROWS)],
                        accum.at[pl.ds(row0, OWN_ROWS)])

        @pl.when(wid == NW - 1)
        def _zero_tail():
            pltpu.sync_copy(zeros_hbm.at[pl.ds(NW * OWN_ROWS, OWN_TAIL)],
                            accum.at[pl.ds(NW * OWN_ROWS, OWN_TAIL)])

        pltpu.sync_copy(pch_hbm, pch_v)
        pltpu.sync_copy(pst_hbm, pst_v)
        nch = pch_v[pl.ds(wid, 1)][0]
        base = pst_v[pl.ds(wid, 1)][0]

        def chunk(i, _):
            off = (base + i) * CH
            pltpu.sync_copy(eid_hbm.at[pl.ds(off, CH)], eid_v)
            pltpu.sync_copy(dstp_hbm.at[pl.ds(off, CH)], idx_v)
            cp = pltpu.async_copy(wm_hbm.at[eid_v], buf_v, sem)
            cp.wait()
            pltpu.sync_copy(buf_v, accum.at[idx_v], add=True)
            return _

        lax.fori_loop(0, nch, chunk, None)
        pltpu.sync_copy(accum.at[pl.ds(row0, OWN_ROWS)],
                        out_hbm.at[pl.ds(row0, OWN_ROWS)])

        @pl.when(wid == NW - 1)
        def _flush_tail():
            pltpu.sync_copy(accum.at[pl.ds(NW * OWN_ROWS, OWN_TAIL)],
                            out_hbm.at[pl.ds(NW * OWN_ROWS, OWN_TAIL)])

    return sc_gather2, sc_scatter_add


def _sc_gather2(a, b, dst, src):
    return _get_sc_kernels()[0](a, b, dst, src)


def _sc_scatter_add(wm, eid_p, dst_p, pch, pst, zeros_nh):
    return _get_sc_kernels()[1](wm, eid_p, dst_p, pch, pst, zeros_nh)


def kernel(x, pos, edge_index, edge_attr, node_dist, params):
    x2d = x.astype(jnp.int32).reshape(N, 1)
    dst = edge_index[1].astype(jnp.int32)
    src = edge_index[0].astype(jnp.int32)

    def lins(p):
        out = []
        for w, b in p["lins"]:
            out.append(w)
            out.append(b.reshape(1, -1))
        return out

    embp = jnp.zeros((16, 16), _f32).at[:9].set(params["embed"])

    bucket = jnp.minimum(dst // OWN_ROWS, NW - 1)
    perm = jnp.argsort(bucket, stable=True).astype(jnp.int32)
    cnts = jnp.bincount(bucket, length=NW)
    pch = ((cnts + CH - 1) // CH).astype(jnp.int32)
    pst = jnp.concatenate([jnp.zeros((1,), jnp.int32),
                           jnp.cumsum(pch)[:-1].astype(jnp.int32)])
    starts = jnp.concatenate([jnp.zeros((1,), cnts.dtype),
                              jnp.cumsum(cnts)[:-1]])
    bs = bucket[perm]
    slot = (pst[bs] * CH + jnp.arange(E) - starts[bs]).astype(jnp.int32)
    eid_p = jnp.zeros((LPAD,), jnp.int32).at[slot].set(perm)
    dst_p = jnp.full((LPAD,), N, jnp.int32).at[slot].set(dst[perm])

    nf = _ln(_node_enc(x2d, pos, embp, *lins(params["node_in"])),
             *params["node_in"]["ln"])
    ef = _ln(_edge_enc(edge_attr, *lins(params["edge_in"])),
             *params["edge_in"]["ln"])

    zeros_nh = jnp.zeros((N, H), _f32)

    for lp in params["mp"]:
        ga, gb = _sc_gather2(nf, nf, dst, src)
        msg = _ln(_edge_dots(ga, gb, ef, *lins(lp["lin_edge"])),
                  *lp["lin_edge"]["ln"])
        aggr, = _sc_scatter_add(msg * node_dist, eid_p, dst_p, pch, pst,
                                zeros_nh)
        upd = _ln(_node_dots(nf, aggr, *lins(lp["lin_node"])),
                  *lp["lin_node"]["ln"])
        ef = ef + msg
        nf = nf + upd

    return _decoder(nf, *lins(params["node_out"]))


# speedup vs baseline: 3.7787x; 1.0452x over previous
import functools

import jax
import jax.numpy as jnp
from jax import lax
from jax.experimental import pallas as pl
from jax.experimental.pallas import tpu as pltpu
from jax.experimental.pallas import tpu_sc as plsc

N = 10000
E = 160000
H = 128
NC, NS = 2, 16
NW = NC * NS
CH = 128
NCHUNK = E // CH
OWN_ROWS = 312
OWN_TAIL = N - NW * OWN_ROWS
LPAD = E + NW * CH

BN = 5000
BE = 8000

_f32 = jnp.float32


def _full(shape):
    return pl.BlockSpec(shape, lambda i: (0,) * len(shape))


def _rows(block):
    return pl.BlockSpec(block, lambda i: (i,) + (0,) * (len(block) - 1))


def _ln(x, g, be):
    mu = jnp.mean(x, axis=-1, keepdims=True)
    var = jnp.var(x, axis=-1, keepdims=True)
    return (x - mu) / jnp.sqrt(var + 1e-5) * g + be


def _node_enc_body(x_ref, pos_ref, embp_ref, w1_ref, b1_ref,
                   w2_ref, b2_ref, w3_ref, b3_ref, h_ref):
    oh = (x_ref[...] == lax.broadcasted_iota(jnp.int32, (BN, 16), 1)).astype(_f32)
    e = jnp.dot(oh, embp_ref[...], preferred_element_type=_f32,
                precision=lax.Precision.HIGHEST)
    cat = jnp.concatenate([e, pos_ref[...]], axis=1)
    h = jnp.maximum(jnp.dot(cat, w1_ref[...], preferred_element_type=_f32)
                    + b1_ref[...], 0.0)
    h = jnp.maximum(jnp.dot(h, w2_ref[...], preferred_element_type=_f32)
                    + b2_ref[...], 0.0)
    h_ref[...] = jnp.dot(h, w3_ref[...], preferred_element_type=_f32) + b3_ref[...]


def _node_enc(x2d, pos, embp, w1, b1, w2, b2, w3, b3):
    return pl.pallas_call(
        _node_enc_body,
        grid=(N // BN,),
        in_specs=[
            _rows((BN, 1)), _rows((BN, 14)),
            _full((16, 16)), _full((30, H)), _full((1, H)),
            _full((H, H)), _full((1, H)), _full((H, H)), _full((1, H)),
        ],
        out_specs=[_rows((BN, H))],
        out_shape=[jax.ShapeDtypeStruct((N, H), _f32)],
    )(x2d, pos, embp, w1, b1, w2, b2, w3, b3)[0]


def _edge_enc_body(ea_ref, w1_ref, b1_ref, w2_ref, b2_ref, w3_ref, b3_ref, h_ref):
    h = jnp.maximum(jnp.dot(ea_ref[...], w1_ref[...], preferred_element_type=_f32)
                    + b1_ref[...], 0.0)
    h = jnp.maximum(jnp.dot(h, w2_ref[...], preferred_element_type=_f32)
                    + b2_ref[...], 0.0)
    h_ref[...] = jnp.dot(h, w3_ref[...], preferred_element_type=_f32) + b3_ref[...]


def _edge_enc(ea, w1, b1, w2, b2, w3, b3):
    return pl.pallas_call(
        _edge_enc_body,
        grid=(E // BE,),
        in_specs=[
            _rows((BE, 3)),
            _full((3, H)), _full((1, H)), _full((H, H)), _full((1, H)),
            _full((H, H)), _full((1, H)),
        ],
        out_specs=[_rows((BE, H))],
        out_shape=[jax.ShapeDtypeStruct((E, H), _f32)],
    )(ea, w1, b1, w2, b2, w3, b3)[0]


def _edge_dots_body(ga_ref, gb_ref, ef_ref, w1_ref, b1_ref,
                    w2_ref, b2_ref, w3_ref, b3_ref, h_ref):
    cat = jnp.concatenate([ga_ref[...], gb_ref[...], ef_ref[...]], axis=1)
    u = jnp.maximum(jnp.dot(cat, w1_ref[...], preferred_element_type=_f32)
                    + b1_ref[...], 0.0)
    h = jnp.maximum(jnp.dot(u, w2_ref[...], preferred_element_type=_f32)
                    + b2_ref[...], 0.0)
    h_ref[...] = jnp.dot(h, w3_ref[...], preferred_element_type=_f32) + b3_ref[...]


def _edge_dots(ga, gb, ef, w1, b1, w2, b2, w3, b3):
    return pl.pallas_call(
        _edge_dots_body,
        grid=(E // BE,),
        in_specs=[
            _rows((BE, H)), _rows((BE, H)), _rows((BE, H)),
            _full((3 * H, H)), _full((1, H)), _full((H, H)), _full((1, H)),
            _full((H, H)), _full((1, H)),
        ],
        out_specs=[_rows((BE, H))],
        out_shape=[jax.ShapeDtypeStruct((E, H), _f32)],
    )(ga, gb, ef, w1, b1, w2, b2, w3, b3)[0]


def _node_dots_body(nf_ref, ag_ref, w1_ref, b1_ref,
                    w2_ref, b2_ref, w3_ref, b3_ref, h_ref):
    cat = jnp.concatenate([nf_ref[...], ag_ref[...]], axis=1)
    u = jnp.maximum(jnp.dot(cat, w1_ref[...], preferred_element_type=_f32)
                    + b1_ref[...], 0.0)
    h = jnp.maximum(jnp.dot(u, w2_ref[...], preferred_element_type=_f32)
                    + b2_ref[...], 0.0)
    h_ref[...] = jnp.dot(h, w3_ref[...], preferred_element_type=_f32) + b3_ref[...]


def _node_dots(nf, aggr, w1, b1, w2, b2, w3, b3):
    return pl.pallas_call(
        _node_dots_body,
        grid=(N // BN,),
        in_specs=[
            _rows((BN, H)), _rows((BN, H)),
            _full((2 * H, H)), _full((1, H)), _full((H, H)), _full((1, H)),
            _full((H, H)), _full((1, H)),
        ],
        out_specs=[_rows((BN, H))],
        out_shape=[jax.ShapeDtypeStruct((N, H), _f32)],
    )(nf, aggr, w1, b1, w2, b2, w3, b3)[0]


def _decoder_body(nf_ref, w1_ref, b1_ref, w2_ref, b2_ref, w3_ref, b3_ref, out_ref):
    h = jnp.maximum(jnp.dot(nf_ref[...], w1_ref[...], preferred_element_type=_f32)
                    + b1_ref[...], 0.0)
    h = jnp.maximum(jnp.dot(h, w2_ref[...], preferred_element_type=_f32)
                    + b2_ref[...], 0.0)
    out_ref[...] = jnp.dot(h, w3_ref[...], preferred_element_type=_f32) + b3_ref[...]


def _decoder(nf, w1, b1, w2, b2, w3, b3):
    return pl.pallas_call(
        _decoder_body,
        grid=(N // BN,),
        in_specs=[
            _rows((BN, H)),
            _full((H, H)), _full((1, H)), _full((H, H)), _full((1, H)),
            _full((H, 2)), _full((1, 2)),
        ],
        out_specs=[_rows((BN, 2))],
        out_shape=[jax.ShapeDtypeStruct((N, 2), _f32)],
    )(nf, w1, b1, w2, b2, w3, b3)[0]


@functools.lru_cache(maxsize=None)
def _get_sc_kernels():
    mesh = plsc.VectorSubcoreMesh(core_axis_name="c", subcore_axis_name="s",
                                  num_cores=NC, num_subcores=NS)

    @functools.partial(
        pl.kernel,
        mesh=mesh,
        out_type=[jax.ShapeDtypeStruct((E, H), _f32)] * 2,
        scratch_types=[
            pltpu.VMEM((CH,), jnp.int32),
            pltpu.VMEM((CH,), jnp.int32),
            pltpu.VMEM((CH, H), _f32),
            pltpu.VMEM((CH, H), _f32),
            pltpu.SemaphoreType.DMA,
            pltpu.SemaphoreType.DMA,
        ],
    )
    def sc_gather2(a_hbm, b_hbm, dst_hbm, src_hbm, oa_hbm, ob_hbm,
                   idx_a, idx_b, buf_a, buf_b, sem_a, sem_b):
        wid = lax.axis_index("s") * NC + lax.axis_index("c")
        nch = 39 + (wid < NCHUNK - 39 * NW).astype(jnp.int32)

        def chunk(i, _):
            off = (wid + i * NW) * CH
            pltpu.sync_copy(dst_hbm.at[pl.ds(off, CH)], idx_a)
            pltpu.sync_copy(src_hbm.at[pl.ds(off, CH)], idx_b)
            cp_a = pltpu.async_copy(a_hbm.at[idx_a], buf_a, sem_a)
            cp_b = pltpu.async_copy(b_hbm.at[idx_b], buf_b, sem_b)
            cp_a.wait()
            cp_b.wait()
            pltpu.sync_copy(buf_a, oa_hbm.at[pl.ds(off, CH)])
            pltpu.sync_copy(buf_b, ob_hbm.at[pl.ds(off, CH)])
            return _

        lax.fori_loop(0, nch, chunk, None)

    @functools.partial(
        pl.kernel,
        mesh=mesh,
        out_type=[jax.ShapeDtypeStruct((N, H), _f32)],
        scratch_types=[
            pltpu.VMEM((NW,), jnp.int32),
            pltpu.VMEM((NW,), jnp.int32),
            pltpu.VMEM((CH,), jnp.int32),
            pltpu.VMEM((CH,), jnp.int32),
            pltpu.VMEM((CH, H), _f32),
            pltpu.VMEM_SHARED((N + 8, H), _f32),
            pltpu.SemaphoreType.DMA,
        ],
    )
    def sc_scatter_add(wm_hbm, eid_hbm, dstp_hbm, pch_hbm, pst_hbm, zeros_hbm,
                       out_hbm, pch_v, pst_v, eid_v, idx_v, buf_v, accum, sem):
        cid = lax.axis_index("c")
        sid = lax.axis_index("s")
        wid = sid * NC + cid
        row0 = wid * OWN_ROWS
        pltpu.sync_copy(zeros_hbm.at[pl.ds(row0, OWN_ROWS)],
                        accum.at[pl.ds(row0, OWN_ROWS)])

        @pl.when(wid == NW - 1)
        def _zero_tail():
            pltpu.sync_copy(zeros_hbm.at[pl.ds(NW * OWN_ROWS, OWN_TAIL)],
                            accum.at[pl.ds(NW * OWN_ROWS, OWN_TAIL)])

        pltpu.sync_copy(pch_hbm, pch_v)
        pltpu.sync_copy(pst_hbm, pst_v)
        nch = pch_v[pl.ds(wid, 1)][0]
        base = pst_v[pl.ds(wid, 1)][0]

        def chunk(i, _):
            off = (base + i) * CH
            pltpu.sync_copy(eid_hbm.at[pl.ds(off, CH)], eid_v)
            pltpu.sync_copy(dstp_hbm.at[pl.ds(off, CH)], idx_v)
            cp = pltpu.async_copy(wm_hbm.at[eid_v], buf_v, sem)
            cp.wait()
            pltpu.sync_copy(buf_v, accum.at[idx_v], add=True)
            return _

        lax.fori_loop(0, nch, chunk, None)
        pltpu.sync_copy(accum.at[pl.ds(row0, OWN_ROWS)],
                        out_hbm.at[pl.ds(row0, OWN_ROWS)])

        @pl.when(wid == NW - 1)
        def _flush_tail():
            pltpu.sync_copy(accum.at[pl.ds(NW * OWN_ROWS, OWN_TAIL)],
                            out_hbm.at[pl.ds(NW * OWN_ROWS, OWN_TAIL)])

    return sc_gather2, sc_scatter_add


def _sc_gather2(a, b, dst, src):
    return _get_sc_kernels()[0](a, b, dst, src)


def _sc_scatter_add(wm, eid_p, dst_p, pch, pst, zeros_nh):
    return _get_sc_kernels()[1](wm, eid_p, dst_p, pch, pst, zeros_nh)


def kernel(x, pos, edge_index, edge_attr, node_dist, params):
    x2d = x.astype(jnp.int32).reshape(N, 1)
    dst = edge_index[1].astype(jnp.int32)
    src = edge_index[0].astype(jnp.int32)

    def lins(p):
        out = []
        for w, b in p["lins"]:
            out.append(w)
            out.append(b.reshape(1, -1))
        return out

    embp = jnp.zeros((16, 16), _f32).at[:9].set(params["embed"])

    bucket = jnp.minimum(dst // OWN_ROWS, NW - 1)
    perm = jnp.argsort(bucket, stable=True).astype(jnp.int32)
    cnts = jnp.bincount(bucket, length=NW)
    pch = ((cnts + CH - 1) // CH).astype(jnp.int32)
    pst = jnp.concatenate([jnp.zeros((1,), jnp.int32),
                           jnp.cumsum(pch)[:-1].astype(jnp.int32)])
    starts = jnp.concatenate([jnp.zeros((1,), cnts.dtype),
                              jnp.cumsum(cnts)[:-1]])
    bs = bucket[perm]
    slot = (pst[bs] * CH + jnp.arange(E) - starts[bs]).astype(jnp.int32)
    eid_p = jnp.zeros((LPAD,), jnp.int32).at[slot].set(perm)
    dst_p = jnp.full((LPAD,), N, jnp.int32).at[slot].set(dst[perm])

    nf = _ln(_node_enc(x2d, pos, embp, *lins(params["node_in"])),
             *params["node_in"]["ln"])
    ef = _ln(_edge_enc(edge_attr, *lins(params["edge_in"])),
             *params["edge_in"]["ln"])

    zeros_nh = jnp.zeros((N, H), _f32)

    for lp in params["mp"]:
        ga, gb = _sc_gather2(nf, nf, dst, src)
        msg = _ln(_edge_dots(ga, gb, ef, *lins(lp["lin_edge"])),
                  *lp["lin_edge"]["ln"])
        aggr, = _sc_scatter_add(msg * node_dist, eid_p, dst_p, pch, pst,
                                zeros_nh)
        upd = _ln(_node_dots(nf, aggr, *lins(lp["lin_node"])),
                  *lp["lin_node"]["ln"])
        ef = ef + msg
        nf = nf + upd

    return _decoder(nf, *lins(params["node_out"]))
